# baseline (speedup 1.0000x reference)
import functools

import jax
import jax.numpy as jnp
from jax import lax
from jax.experimental import pallas as pl
from jax.experimental.pallas import tpu as pltpu
from jax.experimental.pallas import tpu_sc as plsc

NC, NS = 2, 16
NW = NC * NS
LANES = 16


def _make_kernel(N, D, Dp, C):
    nchunk = N // (NW * C)
    mesh = plsc.VectorSubcoreMesh(
        core_axis_name="c", subcore_axis_name="s",
        num_cores=NC, num_subcores=NS)

    starts = list(range(0, D - LANES + 1, LANES))
    if D % LANES:
        starts.append(D - LANES)

    @functools.partial(
        pl.kernel,
        out_type=jax.ShapeDtypeStruct((N, D), jnp.float32),
        mesh=mesh,
        compiler_params=pltpu.CompilerParams(use_tc_tiling_on_sc=False),
        scratch_types=(
            [pltpu.VMEM((nchunk, C), jnp.int32) for _ in range(6)]
            + [pltpu.VMEM((C, Dp), jnp.float32) for _ in range(6)]
            + [pltpu.VMEM((C, D), jnp.float32)]
            + [pltpu.SemaphoreType.DMA]
        ),
    )
    def k(tok_i, pos_i, seg_i, col_i, row_i, rank_i,
          word_t, pos_t, seg_t, col_t, row_t, rank_t,
          out_hbm,
          tok_v, pos_v, seg_v, col_v, row_v, rank_v,
          bw, bp, bs, bc, br, brk, bo,
          sem):
        wid = lax.axis_index("s") * NC + lax.axis_index("c")
        base = wid * (nchunk * C)

        idx_pairs = ((tok_i, tok_v), (pos_i, pos_v), (seg_i, seg_v),
                     (col_i, col_v), (row_i, row_v), (rank_i, rank_v))
        cps = [pltpu.async_copy(src.at[wid], dst, sem)
               for src, dst in idx_pairs]
        for cp in cps:
            cp.wait()

        tables = (word_t, pos_t, seg_t, col_t, row_t, rank_t)
        idxs = (tok_v, pos_v, seg_v, col_v, row_v, rank_v)
        bufs = (bw, bp, bs, bc, br, brk)

        def chunk_body(j, carry):
            gcps = [pltpu.async_copy(tbl.at[iv.at[j]], buf, sem)
                    for tbl, iv, buf in zip(tables, idxs, bufs)]
            for cp in gcps:
                cp.wait()

            def row_body(r, carry2):
                for st in starts:
                    sl = pl.ds(st, LANES)
                    bo[r, sl] = (((((bw[r, sl] + bp[r, sl]) + bs[r, sl])
                                   + bc[r, sl]) + br[r, sl]) + brk[r, sl])
                return carry2

            lax.fori_loop(0, C, row_body, 0)
            pltpu.sync_copy(bo, out_hbm.at[pl.ds(base + j * C, C)])
            return carry

        lax.fori_loop(0, nchunk, chunk_body, 0)

    return k


def kernel(token_ids, seg_ids, col_ids, row_ids, rank_ids,
           word_table, pos, seg_id, col_id, row_id, rank_id):
    B, L = token_ids.shape
    D = word_table.shape[1]
    N = B * L
    C = 40
    assert N % (NW * C) == 0
    nchunk = N // (NW * C)

    Dp = (D + LANES - 1) // LANES * LANES

    def pad_t(t):
        return jnp.pad(t, ((0, 0), (0, Dp - t.shape[1])))

    pos_ids = lax.broadcasted_iota(jnp.int32, (B, L), 1)

    def shape_idx(a):
        return a.reshape(NW, nchunk, C)

    out = _make_kernel(N, D, Dp, C)(
        shape_idx(token_ids), shape_idx(pos_ids), shape_idx(seg_ids),
        shape_idx(col_ids), shape_idx(row_ids), shape_idx(rank_ids),
        pad_t(word_table), pad_t(pos), pad_t(seg_id),
        pad_t(col_id), pad_t(row_id), pad_t(rank_id))
    return out.reshape(B, L, D)
